# baseline (speedup 1.0000x reference)
import dataclasses
import functools

import jax
import jax.numpy as jnp
from jax import lax
from jax.experimental import pallas as pl
from jax.experimental.pallas import tpu as pltpu
from jax.experimental.pallas import tpu_sc as plsc

_NW = 32
_WIN = 896
_L = 16


def _sc_compiler_params():
    cp = pltpu.CompilerParams(use_tc_tiling_on_sc=False)
    if "needs_layout_passes" in pltpu.CompilerParams.__dataclass_fields__:
        cp = dataclasses.replace(cp, needs_layout_passes=False)
    return cp


def _make_sampler(N, C, H, W, P):
    mesh = plsc.VectorSubcoreMesh(core_axis_name="c", subcore_axis_name="s")
    NP = N * P
    nwin_tot = NP // _WIN
    nprep = nwin_tot // _NW
    nwin_b = P // _WIN
    cq = C // 4
    npair = cq // 2

    @functools.partial(
        pl.kernel,
        mesh=mesh,
        out_type=[
            jax.ShapeDtypeStruct((N, C, P), jnp.float32),
            jax.ShapeDtypeStruct((nwin_tot, 5, _WIN), jnp.int32),
        ],
        compiler_params=_sc_compiler_params(),
        scratch_types=[
            pltpu.VMEM((P,), jnp.float32),
            pltpu.VMEM((P,), jnp.float32),
            pltpu.VMEM((2, 5, _WIN), jnp.int32),
            pltpu.VMEM((2, 2, _WIN), jnp.float32),
            pltpu.VMEM((_WIN,), jnp.float32),
            pltpu.VMEM((_WIN,), jnp.float32),
            pltpu.SemaphoreType.DMA,
            pltpu.SemaphoreType.DMA,
            pltpu.SemaphoreType.DMA,
            pltpu.SemaphoreType.DMA,
            pltpu.SemaphoreType.DMA,
        ],
    )
    def sampler(x_hbm, gx_hbm, gy_hbm, out_hbm, idx_hbm,
                imga, imgb, idx2, ov2, gxv, gyv,
                insem0, insem1, outsem0, outsem1, imgsem):
        wid = lax.axis_index("c") * 16 + lax.axis_index("s")
        n = wid // 4
        q = wid % 4
        insems = (insem0, insem1)
        outsems = (outsem0, outsem1)

        @pl.loop(0, nprep)
        def _prepwin(w):
            gwin = wid * nprep + w
            start = gwin * _WIN
            pltpu.sync_copy(gx_hbm.at[pl.ds(start, _WIN)], gxv)
            pltpu.sync_copy(gy_hbm.at[pl.ds(start, _WIN)], gyv)

            @pl.loop(0, _WIN, step=_L)
            def _prep16(i):
                s = pl.ds(i, _L)
                ix = (gxv[s] + 1.0) * (W * 0.5) - 0.5
                iy = (gyv[s] + 1.0) * (H * 0.5) - 0.5
                ix0 = (ix + 1024.0).astype(jnp.int32) - 1024
                iy0 = (iy + 1024.0).astype(jnp.int32) - 1024
                wx1 = ix - ix0.astype(jnp.float32)
                wy1 = iy - iy0.astype(jnp.float32)
                wx0 = 1.0 - wx1
                wy0 = 1.0 - wy1
                vx0 = (ix0 >= 0) & (ix0 <= W - 1)
                vx1 = (ix0 >= -1) & (ix0 <= W - 2)
                vy0 = (iy0 >= 0) & (iy0 <= H - 1)
                vy1 = (iy0 >= -1) & (iy0 <= H - 2)
                cx0 = jnp.clip(ix0, 0, W - 1)
                cx1 = jnp.clip(ix0 + 1, 0, W - 1)
                r0 = jnp.clip(iy0, 0, H - 1) * W
                r1 = jnp.clip(iy0 + 1, 0, H - 1) * W
                idx2[0, 0, s] = ((r0 + cx0) | ((cx1 - cx0) << 16)
                                 | ((r1 - r0) // W << 17))
                zero = jnp.zeros((_L,), jnp.float32)
                w00 = jnp.where(vx0 & vy0, wx0 * wy0, zero)
                w01 = jnp.where(vx1 & vy0, wx1 * wy0, zero)
                w10 = jnp.where(vx0 & vy1, wx0 * wy1, zero)
                w11 = jnp.where(vx1 & vy1, wx1 * wy1, zero)
                idx2[0, 1, s] = plsc.bitcast(w00, jnp.int32)
                idx2[0, 2, s] = plsc.bitcast(w01, jnp.int32)
                idx2[0, 3, s] = plsc.bitcast(w10, jnp.int32)
                idx2[0, 4, s] = plsc.bitcast(w11, jnp.int32)

            pltpu.sync_copy(idx2.at[0], idx_hbm.at[gwin])

        plsc.subcore_barrier()

        def fire_in(gw, b):
            pltpu.async_copy(idx_hbm.at[gw], idx2.at[b], insems[b])

        def wait_in(b):
            pltpu.make_async_copy(idx_hbm.at[0], idx2.at[b],
                                  insems[b]).wait()

        def wait_out(b):
            pltpu.make_async_copy(
                ov2.at[b], out_hbm.at[0, pl.ds(0, 2), pl.ds(0, _WIN)],
                outsems[b]).wait()

        @pl.loop(0, npair)
        def _pair(t):
            c0 = q * cq + 2 * t
            pltpu.async_copy(x_hbm.at[n, c0], imga, imgsem)
            pltpu.async_copy(x_hbm.at[n, c0 + 1], imgb, imgsem)
            pltpu.make_async_copy(x_hbm.at[0, 0], imga, imgsem).wait()
            pltpu.make_async_copy(x_hbm.at[0, 0], imgb, imgsem).wait()
            fire_in(n * nwin_b, 0)

            @pl.loop(0, nwin_b // 2)
            def _wpair(wi2):
                for b in (0, 1):
                    wi = wi2 * 2 + b

                    @pl.when(wi < nwin_b - 1)
                    def _prefetch():
                        fire_in(n * nwin_b + wi + 1, 1 - b)

                    wait_in(b)

                    @pl.when(t * nwin_b + wi >= 2)
                    def _reclaim():
                        wait_out(b)

                    @pl.loop(0, _WIN, step=_L)
                    def _group(i):
                        for u in range(1):
                            s = pl.ds(i + u * _L, _L)
                            v = idx2[b, 0, s]
                            i00 = v & 0xFFFF
                            i01 = i00 + ((v >> 16) & 1)
                            i10 = i00 + ((v >> 17) & 1) * W
                            i11 = i10 + (i01 - i00)
                            w00 = plsc.bitcast(idx2[b, 1, s], jnp.float32)
                            w01 = plsc.bitcast(idx2[b, 2, s], jnp.float32)
                            w10 = plsc.bitcast(idx2[b, 3, s], jnp.float32)
                            w11 = plsc.bitcast(idx2[b, 4, s], jnp.float32)
                            ov2[b, 0, s] = (
                                plsc.load_gather(imga, [i00]) * w00
                                + plsc.load_gather(imga, [i01]) * w01
                                + plsc.load_gather(imga, [i10]) * w10
                                + plsc.load_gather(imga, [i11]) * w11)
                            ov2[b, 1, s] = (
                                plsc.load_gather(imgb, [i00]) * w00
                                + plsc.load_gather(imgb, [i01]) * w01
                                + plsc.load_gather(imgb, [i10]) * w10
                                + plsc.load_gather(imgb, [i11]) * w11)

                    pltpu.async_copy(
                        ov2.at[b],
                        out_hbm.at[n, pl.ds(c0, 2), pl.ds(wi * _WIN, _WIN)],
                        outsems[b])

        wait_out(0)
        wait_out(1)

    return sampler


def kernel(x, grid):
    N, C, H, W = x.shape
    P = H * W
    NP = N * P
    gx = grid[..., 0].reshape(NP)
    gy = grid[..., 1].reshape(NP)
    out, _ = _make_sampler(N, C, H, W, P)(x.reshape(N, C, P), gx, gy)
    return out.reshape(N, C, H, W)


# speedup vs baseline: 1.0242x; 1.0242x over previous
import dataclasses
import functools

import jax
import jax.numpy as jnp
from jax import lax
from jax.experimental import pallas as pl
from jax.experimental.pallas import tpu as pltpu
from jax.experimental.pallas import tpu_sc as plsc

_NW = 32
_WIN = 896
_L = 16


def _sc_compiler_params():
    cp = pltpu.CompilerParams(use_tc_tiling_on_sc=False)
    if "needs_layout_passes" in pltpu.CompilerParams.__dataclass_fields__:
        cp = dataclasses.replace(cp, needs_layout_passes=False)
    return cp


def _make_sampler(N, C, H, W, P):
    mesh = plsc.VectorSubcoreMesh(core_axis_name="c", subcore_axis_name="s")
    NP = N * P
    nwin_tot = NP // _WIN
    nprep = nwin_tot // _NW
    nwin_b = P // _WIN
    cq = C // 4
    npair = cq // 2

    @functools.partial(
        pl.kernel,
        mesh=mesh,
        out_type=[
            jax.ShapeDtypeStruct((N, C, P), jnp.float32),
            jax.ShapeDtypeStruct((nwin_tot, 8, _WIN), jnp.int32),
        ],
        compiler_params=_sc_compiler_params(),
        scratch_types=[
            pltpu.VMEM((P,), jnp.float32),
            pltpu.VMEM((P,), jnp.float32),
            pltpu.VMEM((2, 8, _WIN), jnp.int32),
            pltpu.VMEM((2, 2, _WIN), jnp.float32),
            pltpu.VMEM((_WIN,), jnp.float32),
            pltpu.VMEM((_WIN,), jnp.float32),
            pltpu.SemaphoreType.DMA,
            pltpu.SemaphoreType.DMA,
            pltpu.SemaphoreType.DMA,
            pltpu.SemaphoreType.DMA,
            pltpu.SemaphoreType.DMA,
        ],
    )
    def sampler(x_hbm, gx_hbm, gy_hbm, out_hbm, idx_hbm,
                imga, imgb, idx2, ov2, gxv, gyv,
                insem0, insem1, outsem0, outsem1, imgsem):
        wid = lax.axis_index("c") * 16 + lax.axis_index("s")
        n = wid // 4
        q = wid % 4
        insems = (insem0, insem1)
        outsems = (outsem0, outsem1)

        @pl.loop(0, nprep)
        def _prepwin(w):
            gwin = wid * nprep + w
            start = gwin * _WIN
            pltpu.sync_copy(gx_hbm.at[pl.ds(start, _WIN)], gxv)
            pltpu.sync_copy(gy_hbm.at[pl.ds(start, _WIN)], gyv)

            @pl.loop(0, _WIN, step=_L)
            def _prep16(i):
                s = pl.ds(i, _L)
                ix = (gxv[s] + 1.0) * (W * 0.5) - 0.5
                iy = (gyv[s] + 1.0) * (H * 0.5) - 0.5
                ix0 = (ix + 1024.0).astype(jnp.int32) - 1024
                iy0 = (iy + 1024.0).astype(jnp.int32) - 1024
                wx1 = ix - ix0.astype(jnp.float32)
                wy1 = iy - iy0.astype(jnp.float32)
                wx0 = 1.0 - wx1
                wy0 = 1.0 - wy1
                vx0 = (ix0 >= 0) & (ix0 <= W - 1)
                vx1 = (ix0 >= -1) & (ix0 <= W - 2)
                vy0 = (iy0 >= 0) & (iy0 <= H - 1)
                vy1 = (iy0 >= -1) & (iy0 <= H - 2)
                cx0 = jnp.clip(ix0, 0, W - 1)
                cx1 = jnp.clip(ix0 + 1, 0, W - 1)
                r0 = jnp.clip(iy0, 0, H - 1) * W
                r1 = jnp.clip(iy0 + 1, 0, H - 1) * W
                idx2[0, 0, s] = r0 + cx0
                idx2[0, 1, s] = r0 + cx1
                idx2[0, 2, s] = r1 + cx0
                idx2[0, 3, s] = r1 + cx1
                zero = jnp.zeros((_L,), jnp.float32)
                w00 = jnp.where(vx0 & vy0, wx0 * wy0, zero)
                w01 = jnp.where(vx1 & vy0, wx1 * wy0, zero)
                w10 = jnp.where(vx0 & vy1, wx0 * wy1, zero)
                w11 = jnp.where(vx1 & vy1, wx1 * wy1, zero)
                idx2[0, 4, s] = plsc.bitcast(w00, jnp.int32)
                idx2[0, 5, s] = plsc.bitcast(w01, jnp.int32)
                idx2[0, 6, s] = plsc.bitcast(w10, jnp.int32)
                idx2[0, 7, s] = plsc.bitcast(w11, jnp.int32)

            pltpu.sync_copy(idx2.at[0], idx_hbm.at[gwin])

        plsc.subcore_barrier()

        def fire_in(gw, b):
            pltpu.async_copy(idx_hbm.at[gw], idx2.at[b], insems[b])

        def wait_in(b):
            pltpu.make_async_copy(idx_hbm.at[0], idx2.at[b],
                                  insems[b]).wait()

        def wait_out(b):
            pltpu.make_async_copy(
                ov2.at[b], out_hbm.at[0, pl.ds(0, 2), pl.ds(0, _WIN)],
                outsems[b]).wait()

        @pl.loop(0, npair)
        def _pair(t):
            c0 = q * cq + 2 * t
            pltpu.async_copy(x_hbm.at[n, c0], imga, imgsem)
            pltpu.async_copy(x_hbm.at[n, c0 + 1], imgb, imgsem)
            pltpu.make_async_copy(x_hbm.at[0, 0], imga, imgsem).wait()
            pltpu.make_async_copy(x_hbm.at[0, 0], imgb, imgsem).wait()
            fire_in(n * nwin_b, 0)

            @pl.loop(0, nwin_b // 2)
            def _wpair(wi2):
                for b in (0, 1):
                    wi = wi2 * 2 + b

                    @pl.when(wi < nwin_b - 1)
                    def _prefetch():
                        fire_in(n * nwin_b + wi + 1, 1 - b)

                    wait_in(b)

                    @pl.when(t * nwin_b + wi >= 2)
                    def _reclaim():
                        wait_out(b)

                    @pl.loop(0, _WIN, step=_L)
                    def _group(i):
                        for u in range(1):
                            s = pl.ds(i + u * _L, _L)
                            i00 = idx2[b, 0, s]
                            i01 = idx2[b, 1, s]
                            i10 = idx2[b, 2, s]
                            i11 = idx2[b, 3, s]
                            w00 = plsc.bitcast(idx2[b, 4, s], jnp.float32)
                            w01 = plsc.bitcast(idx2[b, 5, s], jnp.float32)
                            w10 = plsc.bitcast(idx2[b, 6, s], jnp.float32)
                            w11 = plsc.bitcast(idx2[b, 7, s], jnp.float32)
                            ov2[b, 0, s] = (
                                plsc.load_gather(imga, [i00]) * w00
                                + plsc.load_gather(imga, [i01]) * w01
                                + plsc.load_gather(imga, [i10]) * w10
                                + plsc.load_gather(imga, [i11]) * w11)
                            ov2[b, 1, s] = (
                                plsc.load_gather(imgb, [i00]) * w00
                                + plsc.load_gather(imgb, [i01]) * w01
                                + plsc.load_gather(imgb, [i10]) * w10
                                + plsc.load_gather(imgb, [i11]) * w11)

                    pltpu.async_copy(
                        ov2.at[b],
                        out_hbm.at[n, pl.ds(c0, 2), pl.ds(wi * _WIN, _WIN)],
                        outsems[b])

        wait_out(0)
        wait_out(1)

    return sampler


def kernel(x, grid):
    N, C, H, W = x.shape
    P = H * W
    NP = N * P
    gx = grid[..., 0].reshape(NP)
    gy = grid[..., 1].reshape(NP)
    out, _ = _make_sampler(N, C, H, W, P)(x.reshape(N, C, P), gx, gy)
    return out.reshape(N, C, H, W)
